# Initial kernel scaffold; baseline (speedup 1.0000x reference)
import jax
import jax.numpy as jnp
from jax.experimental import pallas as pl


def kernel(coord, edge_src, edge_dst, c_src, c_dst, r_coord, W_init, b_init, Ws0, Wn0, b0, Ws1, Wn1, b1, W_edge, b_edge, W_score, b_score):
    raise NotImplementedError("write your pallas kernel here")


# speedup vs baseline: 8.1313x; 8.1313x over previous
import functools

import jax
import jax.numpy as jnp
from jax import lax
from jax.experimental import pallas as pl
from jax.experimental.pallas import tpu as pltpu
from jax.experimental.pallas import tpu_sc as plsc

N = 50000
E = 800000
EC = 1024
D = 64
HD = D // 2

NCORE = 2
NTILE = 16
NPAD = 50176
NSTRIPE = NPAD // NTILE
ZROWS = 448
K = 80
ET = E // NTILE
NCH = ET // K

_f32 = jnp.float32


def _segsum_body(nf_hbm, esrc_hbm, edst_hbm, out_hbm,
                 isrc, idst, rows, zbuf, acc, sem_i, sem_g, sem_s):
    c = lax.axis_index("c")
    s = lax.axis_index("s")

    zv = jnp.zeros((16,), _f32)

    @pl.loop(0, ZROWS)
    def _(i):
        for j in range(HD // 16):
            zbuf[i, pl.ds(j * 16, 16)] = zv

    for t in range(NTILE):
        @pl.when(s == t)
        def _():
            for k in range(NSTRIPE // ZROWS):
                pltpu.sync_copy(zbuf,
                                acc.at[pl.ds(t * NSTRIPE + k * ZROWS, ZROWS)])

    ebase = s * ET
    half = c * NPAD

    def fire_idx(ch, b4):
        pltpu.async_copy(esrc_hbm.at[pl.ds(ebase + ch * K, K)], isrc[b4],
                         sem_i[b4])
        pltpu.async_copy(edst_hbm.at[pl.ds(ebase + ch * K, K)], idst[b4],
                         sem_i[b4])

    def wait_idx(b4):
        pltpu.make_async_copy(esrc_hbm.at[pl.ds(0, K)], isrc[b4],
                              sem_i[b4]).wait()
        pltpu.make_async_copy(edst_hbm.at[pl.ds(0, K)], idst[b4],
                              sem_i[b4]).wait()

    def tsrc(b4):
        for j in range(K // 16):
            isrc[b4][pl.ds(j * 16, 16)] = isrc[b4][pl.ds(j * 16, 16)] + half

    def fire_gather(b4):
        pltpu.async_copy(nf_hbm.at[isrc[b4]], rows[b4], sem_g[b4])

    def wait_gather(b4):
        pltpu.make_async_copy(nf_hbm.at[isrc[b4]], rows[b4], sem_g[b4]).wait()

    def fire_scatter(b4, b2):
        pltpu.async_copy(rows[b4], acc.at[idst[b4]], sem_s[b2], add=True)

    def wait_scatter(b4, b2):
        pltpu.make_async_copy(rows[b4], acc.at[idst[b4]], sem_s[b2]).wait()

    plsc.subcore_barrier()

    fire_idx(0, 0)
    fire_idx(1, 1)
    wait_idx(0)
    tsrc(0)
    fire_gather(0)

    @pl.loop(0, NCH - 1, step=4)
    def _(g0):
        for o in range(4):
            ch = g0 + o
            b4 = o % 4

            @pl.when(ch >= 2)
            def _():
                wait_scatter((o + 2) % 4, o % 2)

            @pl.when(ch + 2 < NCH)
            def _():
                fire_idx(ch + 2, (o + 2) % 4)

            nb4 = (o + 1) % 4
            wait_idx(nb4)
            tsrc(nb4)
            fire_gather(nb4)
            wait_gather(b4)
            fire_scatter(b4, o % 2)

    wait_scatter(2, 0)
    wait_gather(0)
    fire_scatter(0, 0)
    wait_scatter(3, 1)
    wait_scatter(0, 0)

    plsc.subcore_barrier()

    for t in range(NTILE):
        @pl.when(s == t)
        def _():
            pltpu.sync_copy(acc.at[pl.ds(t * NSTRIPE, NSTRIPE)],
                            out_hbm.at[pl.ds(c * NPAD + t * NSTRIPE,
                                             NSTRIPE)])


@functools.partial(
    pl.kernel,
    out_type=jax.ShapeDtypeStruct((2 * NPAD, HD), _f32),
    mesh=plsc.VectorSubcoreMesh(core_axis_name="c", subcore_axis_name="s"),
    scratch_types=(
        [pltpu.VMEM((K,), jnp.int32)] * 4
        + [pltpu.VMEM((K,), jnp.int32)] * 4
        + [pltpu.VMEM((K, HD), _f32)] * 4
        + [pltpu.VMEM((ZROWS, HD), _f32),
           pltpu.VMEM_SHARED((NPAD, HD), _f32)]
        + [pltpu.SemaphoreType.DMA] * 10
    ),
    compiler_params=pltpu.CompilerParams(use_tc_tiling_on_sc=False),
)
def _sc_segsum(nf_hbm, esrc_hbm, edst_hbm, out_hbm,
               i0, i1, i2, i3, d0, d1, d2, d3, r0, r1, r2, r3, zbuf, acc,
               si0, si1, si2, si3, sg0, sg1, sg2, sg3, ss0, ss1):
    _segsum_body(nf_hbm, esrc_hbm, edst_hbm, out_hbm,
                 (i0, i1, i2, i3), (d0, d1, d2, d3), (r0, r1, r2, r3),
                 zbuf, acc, (si0, si1, si2, si3), (sg0, sg1, sg2, sg3),
                 (ss0, ss1))


@functools.partial(
    pl.kernel,
    out_type=tuple(jax.ShapeDtypeStruct((EC, HD), _f32) for _ in range(4)),
    mesh=plsc.VectorSubcoreMesh(core_axis_name="c", subcore_axis_name="s"),
    scratch_types=[
        pltpu.VMEM((EC // (NCORE * NTILE),), jnp.int32),
        pltpu.VMEM((EC // (NCORE * NTILE), HD), _f32),
        pltpu.SemaphoreType.DMA,
    ],
    compiler_params=pltpu.CompilerParams(use_tc_tiling_on_sc=False),
)
def _sc_pair(nf_hbm, csrc_hbm, cdst_hbm,
             osl_hbm, osh_hbm, odl_hbm, odh_hbm, idxv, rowsv, sem):
    c = lax.axis_index("c")
    s = lax.axis_index("s")
    per = EC // (NCORE * NTILE)
    base = (s * NCORE + c) * per

    def grab(idx_hbm, lo_hbm, hi_hbm):
        pltpu.sync_copy(idx_hbm.at[pl.ds(base, per)], idxv)
        pltpu.async_copy(nf_hbm.at[idxv], rowsv, sem).wait()
        pltpu.sync_copy(rowsv, lo_hbm.at[pl.ds(base, per)])
        for j in range(per // 16):
            idxv[pl.ds(j * 16, 16)] = idxv[pl.ds(j * 16, 16)] + NPAD
        pltpu.async_copy(nf_hbm.at[idxv], rowsv, sem).wait()
        pltpu.sync_copy(rowsv, hi_hbm.at[pl.ds(base, per)])

    grab(csrc_hbm, osl_hbm, osh_hbm)
    grab(cdst_hbm, odl_hbm, odh_hbm)


_TCROWS = 1568
_GRID = NPAD // _TCROWS


def _half_spec():
    return pl.BlockSpec((2, _TCROWS, HD), lambda i: (0, i, 0))


def _init_body(cb, wb, bb, ob):
    x = cb[...]
    w = wb[...]
    nf = x[:, 0:1] * w[0:1, :] + x[:, 1:2] * w[1:2, :] + bb[...]
    ob[0] = nf[:, :HD]
    ob[1] = nf[:, HD:]


def _tc_init(coord_p, w, b):
    return pl.pallas_call(
        _init_body,
        grid=(_GRID,),
        in_specs=[
            pl.BlockSpec((_TCROWS, 2), lambda i: (i, 0)),
            pl.BlockSpec((2, D), lambda i: (0, 0)),
            pl.BlockSpec((1, D), lambda i: (0, 0)),
        ],
        out_specs=_half_spec(),
        out_shape=jax.ShapeDtypeStruct((2, NPAD, HD), _f32),
    )(coord_p, w, b)


def _layer_body(nfb, aggb, wsb, wnb, bb, ob):
    nl = nfb[0]
    nh = nfb[1]
    ws = wsb[...]
    wn = wnb[...]
    h = jnp.dot(nl, ws[:HD, :], preferred_element_type=_f32)
    h = h + jnp.dot(nh, ws[HD:, :], preferred_element_type=_f32)
    h = h + jnp.dot(aggb[0], wn[:HD, :], preferred_element_type=_f32)
    h = h + jnp.dot(aggb[1], wn[HD:, :], preferred_element_type=_f32)
    h = jnp.maximum(h + bb[...], 0.0)
    ob[0] = nl + h[:, :HD]
    ob[1] = nh + h[:, HD:]


def _tc_layer(nf3, agg3, ws, wn, b):
    return pl.pallas_call(
        _layer_body,
        grid=(_GRID,),
        in_specs=[
            _half_spec(),
            _half_spec(),
            pl.BlockSpec((D, D), lambda i: (0, 0)),
            pl.BlockSpec((D, D), lambda i: (0, 0)),
            pl.BlockSpec((1, D), lambda i: (0, 0)),
        ],
        out_specs=_half_spec(),
        out_shape=jax.ShapeDtypeStruct((2, NPAD, HD), _f32),
    )(nf3, agg3, ws, wn, b)


def _score_body(slb, shb, dlb, dhb, web, beb, wsb, bsb, rcb, wib, bib, ob):
    we = web[...]
    ef = jnp.dot(slb[...], we[0:HD, :], preferred_element_type=_f32)
    ef = ef + jnp.dot(shb[...], we[HD:D, :], preferred_element_type=_f32)
    ef = ef + jnp.dot(dlb[...], we[D:D + HD, :], preferred_element_type=_f32)
    ef = ef + jnp.dot(dhb[...], we[D + HD:2 * D, :],
                      preferred_element_type=_f32)
    ef = jnp.maximum(ef + beb[...], 0.0)
    wsc = wsb[...]
    logits = jnp.sum(ef * wsc[:, D:2 * D], axis=1, keepdims=True)
    rc = rcb[...]
    wi = wib[...]
    remb = rc[:, 0:1] * wi[0:1, :] + rc[:, 1:2] * wi[1:2, :] + bib[...]
    const = jnp.sum(remb * wsc[:, 0:D]) + bsb[0, 0]
    logits = logits + const
    m = jnp.max(logits)
    p = jnp.exp(logits - m)
    ob[...] = p / jnp.sum(p)


def _tc_score(sl, sh, dl, dh, w_edge, b_edge, w_score, b_score,
              r_coord, w_init, b_init):
    return pl.pallas_call(
        _score_body,
        out_shape=jax.ShapeDtypeStruct((EC, 1), _f32),
    )(sl, sh, dl, dh, w_edge, b_edge, w_score, b_score,
      r_coord, w_init, b_init)


def kernel(coord, edge_src, edge_dst, c_src, c_dst, r_coord,
           W_init, b_init, Ws0, Wn0, b0, Ws1, Wn1, b1,
           W_edge, b_edge, W_score, b_score):
    coord_p = jnp.pad(coord, ((0, NPAD - N), (0, 0)))
    esrc = edge_src.astype(jnp.int32)
    edst = edge_dst.astype(jnp.int32)
    b_init_r = b_init.reshape(1, D)

    nf0 = _tc_init(coord_p, W_init, b_init_r)
    agg0 = _sc_segsum(nf0.reshape(2 * NPAD, HD), esrc, edst)
    nf1 = _tc_layer(nf0, agg0.reshape(2, NPAD, HD), Ws0, Wn0, b0.reshape(1, D))
    agg1 = _sc_segsum(nf1.reshape(2 * NPAD, HD), esrc, edst)
    nf2 = _tc_layer(nf1, agg1.reshape(2, NPAD, HD), Ws1, Wn1, b1.reshape(1, D))
    sl, sh, dl, dh = _sc_pair(nf2.reshape(2 * NPAD, HD),
                              c_src.astype(jnp.int32),
                              c_dst.astype(jnp.int32))
    prob = _tc_score(sl, sh, dl, dh, W_edge, b_edge.reshape(1, D),
                     W_score.reshape(1, 2 * D), b_score.reshape(1, 1),
                     r_coord, W_init, b_init_r)
    return prob.reshape(EC)
